# baseline (speedup 1.0000x reference)
import jax
import jax.numpy as jnp
from jax.experimental import pallas as pl
from jax.experimental.pallas import tpu as pltpu

N_TOK = 32 * 576
DIM = 64
K = 1024
BLK = 4608
N_BLKS = N_TOK // BLK
LOSS_SCALE = 1.25 / (N_TOK * DIM)


def _vq_body(x_ref, tab_ref, out_ref, idx_ref, loss_ref, bias_ref):
    i = pl.program_id(0)
    tab = tab_ref[...]

    @pl.when(i == 0)
    def _():
        tabsq = tab * tab
        bias_ref[...] = -0.5 * jax.lax.dot_general(
            jnp.ones((1, DIM), jnp.float32), tabsq,
            (((1,), (1,)), ((), ())),
            preferred_element_type=jnp.float32)
        loss_ref[0, 0] = 0.0

    xb = x_ref[...]
    dots = jax.lax.dot_general(
        xb, tab, (((1,), (1,)), ((), ())),
        preferred_element_type=jnp.float32)
    score = dots + bias_ref[...]
    max_val = jnp.max(score, axis=1, keepdims=True)
    kio = jax.lax.broadcasted_iota(jnp.int32, (BLK, K), 1).astype(jnp.float32)
    first = jnp.where(score == max_val, kio, jnp.float32(K))
    idx_f = jnp.min(first, axis=1, keepdims=True)
    idx_ref[...] = idx_f.astype(jnp.int32)
    oh = jnp.where(kio == idx_f, 1.0, 0.0)
    out_ref[...] = jax.lax.dot_general(
        oh, tab, (((1,), (0,)), ((), ())),
        preferred_element_type=jnp.float32)

    x_sq = jnp.sum(xb * xb)
    loss_ref[0, 0] += (x_sq - 2.0 * jnp.sum(max_val)) * LOSS_SCALE


@jax.jit
def kernel(x, table):
    flat_x = x.reshape(N_TOK, DIM)
    out, idx, loss = pl.pallas_call(
        _vq_body,
        grid=(N_BLKS,),
        in_specs=[
            pl.BlockSpec((BLK, DIM), lambda i: (i, 0)),
            pl.BlockSpec((K, DIM), lambda i: (0, 0)),
        ],
        out_specs=[
            pl.BlockSpec((BLK, DIM), lambda i: (i, 0)),
            pl.BlockSpec((BLK, 1), lambda i: (i, 0)),
            pl.BlockSpec(memory_space=pltpu.SMEM),
        ],
        out_shape=[
            jax.ShapeDtypeStruct((N_TOK, DIM), jnp.float32),
            jax.ShapeDtypeStruct((N_TOK, 1), jnp.int32),
            jax.ShapeDtypeStruct((1, 1), jnp.float32),
        ],
        scratch_shapes=[pltpu.VMEM((1, K), jnp.float32)],
    )(flat_x, table)
    return out.reshape(x.shape), loss[0, 0], idx


# speedup vs baseline: 1.4470x; 1.4470x over previous
import jax
import jax.numpy as jnp
from jax.experimental import pallas as pl
from jax.experimental.pallas import tpu as pltpu

N_BATCH = 32
N_TOK = 32 * 576
TOK = 576
DIM = 64
K = 1024
LOSS_SCALE = 1.25 / (N_TOK * DIM)


def _vq_body(x_ref, tt_ref, tab_ref, out_ref, idx_ref, loss_ref, bias_ref):
    i = pl.program_id(0)
    tt = tt_ref[...]

    @pl.when(i == 0)
    def _():
        bias_ref[...] = -0.5 * jnp.sum(tt * tt, axis=0, keepdims=True)
        loss_ref[0, 0] = 0.0

    xb = x_ref[0]
    dots = jax.lax.dot_general(
        xb, tt, (((0,), (0,)), ((), ())),
        preferred_element_type=jnp.float32)
    score = dots + bias_ref[...]
    max_val = jnp.max(score, axis=1, keepdims=True)
    kio = jax.lax.broadcasted_iota(jnp.int32, (TOK, K), 1).astype(jnp.float32)
    first = jnp.where(score == max_val, kio, jnp.float32(K))
    idx_f = jnp.min(first, axis=1, keepdims=True)
    idx_ref[...] = idx_f.astype(jnp.int32).reshape(1, 1, TOK)
    oh = jnp.where(kio == idx_f, 1.0, 0.0)
    out_ref[0] = jax.lax.dot_general(
        tab_ref[...], oh, (((0,), (1,)), ((), ())),
        preferred_element_type=jnp.float32)

    x_sq = jnp.sum(xb * xb)
    loss_ref[0, 0] += (x_sq - 2.0 * jnp.sum(max_val)) * LOSS_SCALE


@jax.jit
def kernel(x, table):
    xt = x.transpose(0, 2, 1)
    tt = table.T
    out, idx, loss = pl.pallas_call(
        _vq_body,
        grid=(N_BATCH,),
        in_specs=[
            pl.BlockSpec((1, DIM, TOK), lambda i: (i, 0, 0)),
            pl.BlockSpec((DIM, K), lambda i: (0, 0)),
            pl.BlockSpec((K, DIM), lambda i: (0, 0)),
        ],
        out_specs=[
            pl.BlockSpec((1, DIM, TOK), lambda i: (i, 0, 0)),
            pl.BlockSpec((1, 1, TOK), lambda i: (i, 0, 0)),
            pl.BlockSpec(memory_space=pltpu.SMEM),
        ],
        out_shape=[
            jax.ShapeDtypeStruct((N_BATCH, DIM, TOK), jnp.float32),
            jax.ShapeDtypeStruct((N_BATCH, 1, TOK), jnp.int32),
            jax.ShapeDtypeStruct((1, 1), jnp.float32),
        ],
        scratch_shapes=[pltpu.VMEM((1, K), jnp.float32)],
    )(xt, tt, table)
    return out.transpose(0, 2, 1), loss[0, 0], idx.reshape(N_TOK, 1)


# speedup vs baseline: 1.4617x; 1.0101x over previous
import jax
import jax.numpy as jnp
from jax.experimental import pallas as pl
from jax.experimental.pallas import tpu as pltpu

N_BATCH = 32
N_TOK = 32 * 576
TOK = 576
DIM = 64
K = 1024
LOSS_SCALE = 1.25 / (N_TOK * DIM)
DAUG = 72


def _vq_body(x_ref, tt_ref, tab_ref, out_ref, idx_ref, loss_ref, bias_ref,
             aug_ref):
    i = pl.program_id(0)
    tt = tt_ref[...]

    @pl.when(i == 0)
    def _():
        bias_ref[...] = -0.5 * jnp.sum(tt * tt, axis=0, keepdims=True)
        kcol = jax.lax.broadcasted_iota(
            jnp.int32, (K, 1), 0).astype(jnp.float32)
        aug_ref[...] = jnp.concatenate(
            [tab_ref[...], kcol,
             jnp.zeros((K, DAUG - DIM - 1), jnp.float32)], axis=1)
        loss_ref[0, 0] = 0.0

    xb = x_ref[0]
    dots = jax.lax.dot_general(
        xb, tt, (((0,), (0,)), ((), ())),
        preferred_element_type=jnp.float32)
    score = dots + bias_ref[...]
    max_val = jnp.max(score, axis=1, keepdims=True)
    kio = jax.lax.broadcasted_iota(jnp.int32, (TOK, K), 1).astype(jnp.float32)
    first = jnp.where(score == max_val, kio, jnp.float32(K))
    idx_f = jnp.min(first, axis=1, keepdims=True)
    oh = jnp.where(kio == idx_f, 1.0, 0.0)
    out_aug = jax.lax.dot_general(
        aug_ref[...], oh, (((0,), (1,)), ((), ())),
        preferred_element_type=jnp.float32)
    out_ref[0] = out_aug[:DIM]
    idx_ref[...] = out_aug[DIM:DIM + 1].astype(jnp.int32).reshape(1, 1, TOK)

    x_sq = jnp.sum(xb * xb)
    loss_ref[0, 0] += (x_sq - 2.0 * jnp.sum(max_val)) * LOSS_SCALE


@jax.jit
def kernel(x, table):
    xt = x.transpose(0, 2, 1)
    tt = table.T
    out, idx, loss = pl.pallas_call(
        _vq_body,
        grid=(N_BATCH,),
        in_specs=[
            pl.BlockSpec((1, DIM, TOK), lambda i: (i, 0, 0)),
            pl.BlockSpec((DIM, K), lambda i: (0, 0)),
            pl.BlockSpec((K, DIM), lambda i: (0, 0)),
        ],
        out_specs=[
            pl.BlockSpec((1, DIM, TOK), lambda i: (i, 0, 0)),
            pl.BlockSpec((1, 1, TOK), lambda i: (i, 0, 0)),
            pl.BlockSpec(memory_space=pltpu.SMEM),
        ],
        out_shape=[
            jax.ShapeDtypeStruct((N_BATCH, DIM, TOK), jnp.float32),
            jax.ShapeDtypeStruct((N_BATCH, 1, TOK), jnp.int32),
            jax.ShapeDtypeStruct((1, 1), jnp.float32),
        ],
        scratch_shapes=[pltpu.VMEM((1, K), jnp.float32),
                        pltpu.VMEM((K, DAUG), jnp.float32)],
    )(xt, tt, table)
    return out.transpose(0, 2, 1), loss[0, 0], idx.reshape(N_TOK, 1)


# speedup vs baseline: 1.7180x; 1.1754x over previous
import jax
import jax.numpy as jnp
from jax.experimental import pallas as pl
from jax.experimental.pallas import tpu as pltpu

N_BATCH = 32
N_TOK = 32 * 576
TOK = 576
DIM = 64
K = 1024
LOSS_SCALE = 1.25 / (N_TOK * DIM)
DAUG = 72
B_PER_STEP = 4


def _vq_body(x_ref, tt_ref, tab_ref, out_ref, idx_ref, loss_ref, bias_ref,
             aug_ref):
    i = pl.program_id(0)
    tt = tt_ref[...]

    @pl.when(i == 0)
    def _():
        bias_ref[...] = -0.5 * jnp.sum(tt * tt, axis=0, keepdims=True)
        kcol = jax.lax.broadcasted_iota(
            jnp.int32, (K, 1), 0).astype(jnp.float32)
        aug_ref[...] = jnp.concatenate(
            [tab_ref[...], kcol,
             jnp.zeros((K, DAUG - DIM - 1), jnp.float32)], axis=1)
        loss_ref[0, 0] = 0.0

    kio = jax.lax.broadcasted_iota(jnp.int32, (TOK, K), 1).astype(jnp.float32)
    for b in range(B_PER_STEP):
        xb = x_ref[b]
        dots = jax.lax.dot_general(
            xb, tt, (((0,), (0,)), ((), ())),
            preferred_element_type=jnp.float32)
        score = dots + bias_ref[...]
        max_val = jnp.max(score, axis=1, keepdims=True)
        first = jnp.where(score == max_val, kio, jnp.float32(K))
        idx_f = jnp.min(first, axis=1, keepdims=True)
        oh = jnp.where(kio == idx_f, 1.0, 0.0)
        out_aug = jax.lax.dot_general(
            aug_ref[...], oh, (((0,), (1,)), ((), ())),
            preferred_element_type=jnp.float32)
        out_ref[b] = out_aug[:DIM]
        idx_ref[b] = out_aug[DIM:DIM + 1].astype(jnp.int32).reshape(1, TOK)
        x_sq = jnp.sum(xb * xb)
        loss_ref[0, 0] += (x_sq - 2.0 * jnp.sum(max_val)) * LOSS_SCALE


@jax.jit
def kernel(x, table):
    xt = x.transpose(0, 2, 1)
    tt = table.T
    out, idx, loss = pl.pallas_call(
        _vq_body,
        grid=(N_BATCH // B_PER_STEP,),
        in_specs=[
            pl.BlockSpec((B_PER_STEP, DIM, TOK), lambda i: (i, 0, 0)),
            pl.BlockSpec((DIM, K), lambda i: (0, 0)),
            pl.BlockSpec((K, DIM), lambda i: (0, 0)),
        ],
        out_specs=[
            pl.BlockSpec((B_PER_STEP, DIM, TOK), lambda i: (i, 0, 0)),
            pl.BlockSpec((B_PER_STEP, 1, TOK), lambda i: (i, 0, 0)),
            pl.BlockSpec(memory_space=pltpu.SMEM),
        ],
        out_shape=[
            jax.ShapeDtypeStruct((N_BATCH, DIM, TOK), jnp.float32),
            jax.ShapeDtypeStruct((N_BATCH, 1, TOK), jnp.int32),
            jax.ShapeDtypeStruct((1, 1), jnp.float32),
        ],
        scratch_shapes=[pltpu.VMEM((1, K), jnp.float32),
                        pltpu.VMEM((K, DAUG), jnp.float32)],
    )(xt, tt, table)
    return out.transpose(0, 2, 1), loss[0, 0], idx.reshape(N_TOK, 1)
